# baseline (speedup 1.0000x reference)
import functools

import jax
import jax.numpy as jnp
from jax import lax
from jax.experimental import pallas as pl
from jax.experimental.pallas import tpu as pltpu
from jax.experimental.pallas import tpu_sc as plsc

N = 10000
E = 160000
NUM_GRAPHS = 64
OUT_DIM = 256

NC = 2
NS = 16
NW = NC * NS
CW = 128
NCH = 4
B = 80
EPT = E // NS
NB = EPT // B
ST = 5
NBS = NB // ST
WT = 10
RPW = N // WT
ZR = 40

_MESH = plsc.VectorSubcoreMesh(core_axis_name="c", subcore_axis_name="s")
_f32 = jnp.float32


DW = 16


@functools.partial(
    pl.kernel,
    out_type=jax.ShapeDtypeStruct((N, DW), _f32),
    mesh=_MESH,
    scratch_types=[
        pltpu.VMEM((NBS, B), jnp.int32),
        pltpu.VMEM((B, DW), _f32),
        pltpu.VMEM((ZR, DW), _f32),
        pltpu.VMEM_SHARED((N, DW), _f32),
    ],
)
def _deg_kernel(dst_hbm, out_hbm, dst_v, ones_v, zbuf, acc):
    c = lax.axis_index("c")
    s = lax.axis_index("s")

    def fill(r, carry):
        zbuf[r, pl.ds(0, DW)] = jnp.zeros((DW,), _f32)
        return carry

    lax.fori_loop(0, ZR, fill, 0)

    def fill1(r, carry):
        ones_v[r, pl.ds(0, DW)] = jnp.ones((DW,), _f32)
        return carry

    lax.fori_loop(0, B, fill1, 0)

    @pl.when(s < WT)
    def _():
        for z in range(RPW // ZR):
            pltpu.sync_copy(zbuf, acc.at[pl.ds(s * RPW + z * ZR, ZR)])
    plsc.subcore_barrier()

    def batch(jb, carry):
        pltpu.sync_copy(ones_v, acc.at[dst_v.at[jb]], add=True)
        return carry

    for st in range(ST):
        pltpu.sync_copy(dst_hbm.at[s, st], dst_v)
        lax.fori_loop(0, NBS, batch, 0)
    plsc.subcore_barrier()

    @pl.when((c == 0) & (s < WT))
    def _():
        pltpu.sync_copy(acc.at[pl.ds(s * RPW, RPW)],
                        out_hbm.at[pl.ds(s * RPW, RPW)])


@functools.partial(
    pl.kernel,
    out_type=jax.ShapeDtypeStruct((2 * NCH * N, CW), _f32),
    mesh=_MESH,
    scratch_types=[
        pltpu.VMEM((NBS, B), jnp.int32),
        pltpu.VMEM((NBS, B), jnp.int32),
        [pltpu.VMEM((B, CW), _f32)] * 4,
        pltpu.VMEM_SHARED((N, CW), _f32),
        [pltpu.SemaphoreType.DMA] * 4,
        [pltpu.SemaphoreType.DMA] * 4,
    ],
)
def _spmm_kernel(hp_hbm, src_hbm, dst_hbm, out_hbm,
                 src_v, dst_v, rows, acc, gsem, ssem):
    c = lax.axis_index("c")
    s = lax.axis_index("s")

    for j in range(2 * NCH // NC):
        q = c * (2 * NCH // NC) + j
        tbl = hp_hbm.at[pl.ds(q * N, N)]

        def zrow(r, carry):
            for cc in range(CW // 16):
                rows[0][r, pl.ds(cc * 16, 16)] = jnp.zeros((16,), _f32)
            return carry

        lax.fori_loop(0, B, zrow, 0)

        @pl.when(s < WT)
        def _():
            for z in range(RPW // B):
                pltpu.sync_copy(rows[0], acc.at[pl.ds(s * RPW + z * B, B)])
            pltpu.sync_copy(rows[0].at[pl.ds(0, RPW - (RPW // B) * B)],
                            acc.at[pl.ds(s * RPW + (RPW // B) * B,
                                         RPW - (RPW // B) * B)])
        plsc.subcore_barrier()

        for st in range(ST):
            pltpu.sync_copy(src_hbm.at[s, st], src_v)
            pltpu.sync_copy(dst_hbm.at[s, st], dst_v)
            for k in range(4):
                pltpu.async_copy(tbl.at[src_v.at[k]], rows[k], gsem[k])

            def group(jj, carry):
                j0 = 4 * jj
                for k in range(4):
                    pltpu.make_async_copy(
                        tbl.at[src_v.at[0]], rows[k], gsem[k]).wait()
                    pltpu.async_copy(rows[k], acc.at[dst_v.at[j0 + k]],
                                     ssem[k], add=True)
                for k in range(4):
                    pltpu.make_async_copy(
                        rows[k], acc.at[dst_v.at[0]], ssem[k]).wait()
                    nj = j0 + k + 4

                    @pl.when(nj < NBS)
                    def _():
                        pltpu.async_copy(tbl.at[src_v.at[nj]], rows[k],
                                         gsem[k])
                return carry

            lax.fori_loop(0, (NBS - 1) // 4, group, 0)
            pltpu.make_async_copy(tbl.at[src_v.at[0]], rows[0], gsem[0]).wait()
            pltpu.sync_copy(rows[0], acc.at[dst_v.at[NBS - 1]], add=True)

        plsc.subcore_barrier()

        @pl.when(s < WT)
        def _():
            pltpu.sync_copy(acc.at[pl.ds(s * RPW, RPW)],
                            out_hbm.at[pl.ds(q * N + s * RPW, RPW)])
        plsc.subcore_barrier()


RB = 64


@functools.partial(
    pl.kernel,
    out_type=jax.ShapeDtypeStruct((NW, 16, CW), _f32),
    mesh=_MESH,
    scratch_types=[
        pltpu.VMEM((96,), jnp.int32),
        pltpu.VMEM((RB, CW), _f32),
        pltpu.VMEM((16, CW), _f32),
    ],
)
def _segmax_kernel(h2_hbm, starts_hbm, out_hbm, starts_v, blk_v, res_v):
    c = lax.axis_index("c")
    s = lax.axis_index("s")
    wid = s * NC + c
    pltpu.sync_copy(starts_hbm, starts_v)

    def scal(i):
        return starts_v[pl.ds(i, 16)][0]

    for k in range(16):
        tid = k * NW + wid
        g = tid % NUM_GRAPHS
        q = tid // NUM_GRAPHS
        start = scal(g)
        end = scal(g + 1)
        rb0 = 8 * (start // 8)
        nblk = lax.max((end - rb0 + RB - 1) // RB, 0)

        def block(t, accs):
            rb = jnp.minimum(rb0 + t * RB, N - RB)
            pltpu.sync_copy(h2_hbm.at[pl.ds(q * N + rb, RB)], blk_v)

            def row(r, accs):
                keep = (rb + r >= start) & (rb + r < end)
                return tuple(
                    jnp.where(keep,
                              jnp.maximum(accs[i], blk_v[r, pl.ds(i * 16, 16)]),
                              accs[i])
                    for i in range(CW // 16))

            return lax.fori_loop(0, RB, row, accs)

        neg = jnp.full((16,), -jnp.inf, _f32)
        accs = lax.fori_loop(0, nblk, block, (neg,) * (CW // 16))
        for i in range(CW // 16):
            res_v[k, pl.ds(i * 16, 16)] = accs[i]

    pltpu.sync_copy(res_v, out_hbm.at[wid])


BR = 1000
_GRID = N // BR

_cm_spec = pl.BlockSpec((1, NCH, BR, CW), lambda b, i: (b, 0, i, 0))
_dinv_spec = pl.BlockSpec((BR, 1), lambda b, i: (i, 0))
_b_spec = pl.BlockSpec((NCH, 1, CW), lambda b, i: (0, 0, 0))
_w_spec = pl.BlockSpec((512, 512), lambda b, i: (0, 0))


def _deg_reduce_kernel(parts_ref, dinv_ref):
    deg = parts_ref[:, 0:1] + 1.0
    dinv_ref[...] = lax.rsqrt(deg)


def _deg_reduce(parts):
    return pl.pallas_call(
        _deg_reduce_kernel,
        in_specs=[pl.BlockSpec((N, DW), lambda: (0, 0))],
        out_specs=pl.BlockSpec((N, 1), lambda: (0, 0)),
        out_shape=jax.ShapeDtypeStruct((N, 1), _f32),
    )(parts)


def _input_kernel(x_ref, wesm_ref, besm_ref, nat_ref, embp_ref, waa_ref,
                  baa_ref, feats_ref):
    xesm = jnp.dot(x_ref[...], wesm_ref[...],
                   preferred_element_type=_f32) + besm_ref[...]
    embw = jnp.dot(embp_ref[...], waa_ref[...], preferred_element_type=_f32)
    oh = (nat_ref[...] == lax.broadcasted_iota(jnp.int32, (BR, 32), 1)
          ).astype(_f32)
    xaa = jnp.dot(oh, embw, preferred_element_type=_f32) + baa_ref[...]
    h = jax.nn.relu(xaa + xesm)
    xr = jax.nn.relu(xesm)
    for q in range(NCH):
        feats_ref[0, q] = h[:, q * CW:(q + 1) * CW]
        feats_ref[1, q] = xr[:, q * CW:(q + 1) * CW]


def _input_call(x, W_esm, b_esm, nat2, emb_p, W_aa, b_aa):
    return pl.pallas_call(
        _input_kernel,
        grid=(_GRID,),
        in_specs=[
            pl.BlockSpec((BR, 1280), lambda i: (i, 0)),
            pl.BlockSpec((1280, 512), lambda i: (0, 0)),
            pl.BlockSpec((1, 512), lambda i: (0, 0)),
            pl.BlockSpec((BR, 1), lambda i: (i, 0)),
            pl.BlockSpec((32, 96), lambda i: (0, 0)),
            pl.BlockSpec((96, 512), lambda i: (0, 0)),
            pl.BlockSpec((1, 512), lambda i: (0, 0)),
        ],
        out_specs=pl.BlockSpec((2, NCH, BR, CW), lambda i: (0, 0, i, 0)),
        out_shape=jax.ShapeDtypeStruct((2, NCH, N, CW), _f32),
    )(x, W_esm, b_esm.reshape(1, 512), nat2, emb_p, W_aa, b_aa.reshape(1, 512))


def _first_kernel(feat_ref, dinv_ref, w_ref, hp_ref):
    xb = jnp.concatenate([feat_ref[0, q] for q in range(NCH)], axis=-1)
    mm = jnp.dot(xb, w_ref[...], preferred_element_type=_f32) * dinv_ref[...]
    for q in range(NCH):
        hp_ref[0, q] = mm[:, q * CW:(q + 1) * CW]


def _first_mm(feat, dinv, W):
    return pl.pallas_call(
        _first_kernel,
        grid=(2, _GRID),
        in_specs=[_cm_spec, _dinv_spec, _w_spec],
        out_specs=_cm_spec,
        out_shape=jax.ShapeDtypeStruct((2, NCH, N, CW), _f32),
    )(feat, dinv, W)


def _mid_kernel(agg_ref, hp_ref, res_ref, dinv_ref, b_ref, w_ref,
                h_ref, hpn_ref, *, has_res):
    dinv = dinv_ref[...]
    parts = []
    for q in range(NCH):
        t = jax.nn.relu(dinv * (agg_ref[0, q] + hp_ref[0, q]) + b_ref[q])
        if has_res:
            t = res_ref[0, q] + t
        h_ref[0, q] = t
        parts.append(t)
    xb = jnp.concatenate(parts, axis=-1)
    mm = jnp.dot(xb, w_ref[...], preferred_element_type=_f32) * dinv
    for q in range(NCH):
        hpn_ref[0, q] = mm[:, q * CW:(q + 1) * CW]


def _mid_mm(agg, hp, res, dinv, b, W):
    cm = jax.ShapeDtypeStruct((2, NCH, N, CW), _f32)
    has_res = res is not None
    in_specs = [_cm_spec, _cm_spec]
    args = [agg, hp]
    if has_res:
        in_specs.append(_cm_spec)
        args.append(res)
    in_specs += [_dinv_spec, _b_spec, _w_spec]
    args += [dinv, b.reshape(NCH, 1, CW), W]
    body = functools.partial(_mid_kernel, has_res=has_res)
    if not has_res:
        def body(agg_ref, hp_ref, dinv_ref, b_ref, w_ref, h_ref, hpn_ref):
            return _mid_kernel(agg_ref, hp_ref, None, dinv_ref, b_ref, w_ref,
                               h_ref, hpn_ref, has_res=False)
    return pl.pallas_call(
        body,
        grid=(2, _GRID),
        in_specs=in_specs,
        out_specs=[_cm_spec, _cm_spec],
        out_shape=[cm, cm],
    )(*args)


def _few_kernel(agg_ref, hp_ref, res_ref, dinv_ref, b_ref, out_ref):
    dinv = dinv_ref[...]
    for q in range(NCH):
        t = jax.nn.relu(dinv * (agg_ref[0, q] + hp_ref[0, q]) + b_ref[q])
        out_ref[0, q] = res_ref[0, q] + t


def _final_ew(agg, hp, res, dinv, b):
    return pl.pallas_call(
        _few_kernel,
        grid=(2, _GRID),
        in_specs=[_cm_spec, _cm_spec, _cm_spec, _dinv_spec, _b_spec],
        out_specs=_cm_spec,
        out_shape=jax.ShapeDtypeStruct((2, NCH, N, CW), _f32),
    )(agg, hp, res, dinv, b.reshape(NCH, 1, CW))


def _parts_to_g(p_ref, k0):
    half0 = jnp.concatenate(
        [p_ref[:, k0 + 2 * q, :] for q in range(NCH)], axis=1)
    half1 = jnp.concatenate(
        [p_ref[:, k0 + 2 * q + 1, :] for q in range(NCH)], axis=1)
    return jnp.concatenate([half0, half1], axis=0)


def _head_kernel(p_ref, w1_ref, b1_ref, w2_ref, b2_ref, y_ref):
    g = 0.5 * _parts_to_g(p_ref, 0) + 0.5 * _parts_to_g(p_ref, 8)
    z = jax.nn.relu(jnp.dot(g, w1_ref[...], preferred_element_type=_f32)
                    + b1_ref[...])
    y = jnp.dot(z, w2_ref[...], preferred_element_type=_f32) + b2_ref[...]
    y_ref[...] = jax.nn.sigmoid(y)


def _head(parts, W_r1, b_r1, W_r2, b_r2):
    full = lambda shape: pl.BlockSpec(shape, lambda: tuple(0 for _ in shape))
    return pl.pallas_call(
        _head_kernel,
        in_specs=[full((NW, 16, CW)),
                  full((512, 1024)), full((1, 1024)),
                  full((1024, OUT_DIM)), full((1, OUT_DIM))],
        out_specs=full((NUM_GRAPHS, OUT_DIM)),
        out_shape=jax.ShapeDtypeStruct((NUM_GRAPHS, OUT_DIM), _f32),
    )(parts, W_r1, b_r1.reshape(1, 1024), W_r2, b_r2.reshape(1, OUT_DIM))


def kernel(native_x, x, edge_index, batch, emb, W_aa, b_aa, W_esm, b_esm,
           W_g0, b_g0, W_g1, b_g1, W_g2, b_g2, W_r1, b_r1, W_r2, b_r2):
    src = edge_index[0].astype(jnp.int32)
    dst = edge_index[1].astype(jnp.int32)
    src3 = src.reshape(NS, ST, NBS, B)
    dst3 = dst.reshape(NS, ST, NBS, B)

    deg_parts = _deg_kernel(dst3)
    dinv = _deg_reduce(deg_parts)

    emb_p = jnp.zeros((32, 96), _f32).at[:21].set(emb)
    feats = _input_call(x, W_esm, b_esm,
                        native_x.reshape(N, 1).astype(jnp.int32),
                        emb_p, W_aa, b_aa)

    def spmm(hp_cm):
        out = _spmm_kernel(hp_cm.reshape(2 * NCH * N, CW), src3, dst3)
        return out.reshape(2, NCH, N, CW)

    starts = jnp.searchsorted(
        batch.astype(jnp.int32),
        jnp.arange(NUM_GRAPHS + 1, dtype=jnp.int32)).astype(jnp.int32)
    starts96 = jnp.zeros((96,), jnp.int32).at[:NUM_GRAPHS + 1].set(starts)

    hp0 = _first_mm(feats, dinv, W_g0)
    agg0 = spmm(hp0)
    h0, hp1 = _mid_mm(agg0, hp0, None, dinv, b_g0, W_g1)
    agg1 = spmm(hp1)
    h1, hp2 = _mid_mm(agg1, hp1, h0, dinv, b_g1, W_g2)
    agg2 = spmm(hp2)
    h2 = _final_ew(agg2, hp2, h1, dinv, b_g2)
    parts = _segmax_kernel(h2.reshape(2 * NCH * N, CW), starts96)
    return _head(parts, W_r1, b_r1, W_r2, b_r2)


# speedup vs baseline: 1.0523x; 1.0523x over previous
import functools

import jax
import jax.numpy as jnp
from jax import lax
from jax.experimental import pallas as pl
from jax.experimental.pallas import tpu as pltpu
from jax.experimental.pallas import tpu_sc as plsc

N = 10000
E = 160000
NUM_GRAPHS = 64
OUT_DIM = 256

NC = 2
NS = 16
NW = NC * NS
CW = 128
NCH = 4
B = 80
EPT = E // NS
NB = EPT // B
ST = 5
NBS = NB // ST
WT = 10
RPW = N // WT
ZR = 40

_MESH = plsc.VectorSubcoreMesh(core_axis_name="c", subcore_axis_name="s")
_f32 = jnp.float32


DW = 16


@functools.partial(
    pl.kernel,
    out_type=jax.ShapeDtypeStruct((N, DW), _f32),
    mesh=_MESH,
    scratch_types=[
        pltpu.VMEM((NBS, B), jnp.int32),
        pltpu.VMEM((B, DW), _f32),
        pltpu.VMEM((ZR, DW), _f32),
        pltpu.VMEM_SHARED((N, DW), _f32),
    ],
)
def _deg_kernel(dst_hbm, out_hbm, dst_v, ones_v, zbuf, acc):
    c = lax.axis_index("c")
    s = lax.axis_index("s")

    def fill(r, carry):
        zbuf[r, pl.ds(0, DW)] = jnp.zeros((DW,), _f32)
        return carry

    lax.fori_loop(0, ZR, fill, 0)

    def fill1(r, carry):
        ones_v[r, pl.ds(0, DW)] = jnp.ones((DW,), _f32)
        return carry

    lax.fori_loop(0, B, fill1, 0)

    @pl.when(s < WT)
    def _():
        for z in range(RPW // ZR):
            pltpu.sync_copy(zbuf, acc.at[pl.ds(s * RPW + z * ZR, ZR)])
    plsc.subcore_barrier()

    def batch(jb, carry):
        pltpu.sync_copy(ones_v, acc.at[dst_v.at[jb]], add=True)
        return carry

    for st in range(ST):
        pltpu.sync_copy(dst_hbm.at[s, st], dst_v)
        lax.fori_loop(0, NBS, batch, 0)
    plsc.subcore_barrier()

    @pl.when((c == 0) & (s < WT))
    def _():
        pltpu.sync_copy(acc.at[pl.ds(s * RPW, RPW)],
                        out_hbm.at[pl.ds(s * RPW, RPW)])


@functools.partial(
    pl.kernel,
    out_type=jax.ShapeDtypeStruct((NCH * N, CW), _f32),
    mesh=_MESH,
    scratch_types=[
        pltpu.VMEM((NBS, B), jnp.int32),
        pltpu.VMEM((NBS, B), jnp.int32),
        [pltpu.VMEM((B, CW), _f32)] * 4,
        pltpu.VMEM_SHARED((N, CW), _f32),
        [pltpu.SemaphoreType.DMA] * 4,
        [pltpu.SemaphoreType.DMA] * 4,
    ],
)
def _spmm_kernel(hp_hbm, src_hbm, dst_hbm, out_hbm,
                 src_v, dst_v, rows, acc, gsem, ssem):
    c = lax.axis_index("c")
    s = lax.axis_index("s")

    for j in range(NCH // NC):
        q = c * (NCH // NC) + j
        tbl = hp_hbm.at[pl.ds(q * N, N)]

        def zrow(r, carry):
            for cc in range(CW // 16):
                rows[0][r, pl.ds(cc * 16, 16)] = jnp.zeros((16,), _f32)
            return carry

        lax.fori_loop(0, B, zrow, 0)

        @pl.when(s < WT)
        def _():
            for z in range(RPW // B):
                pltpu.sync_copy(rows[0], acc.at[pl.ds(s * RPW + z * B, B)])
            pltpu.sync_copy(rows[0].at[pl.ds(0, RPW - (RPW // B) * B)],
                            acc.at[pl.ds(s * RPW + (RPW // B) * B,
                                         RPW - (RPW // B) * B)])
        plsc.subcore_barrier()

        for st in range(ST):
            pltpu.sync_copy(src_hbm.at[s, st], src_v)
            pltpu.sync_copy(dst_hbm.at[s, st], dst_v)
            for k in range(4):
                pltpu.async_copy(tbl.at[src_v.at[k]], rows[k], gsem[k])

            def group(jj, carry):
                j0 = 4 * jj
                for k in range(4):
                    pltpu.make_async_copy(
                        tbl.at[src_v.at[0]], rows[k], gsem[k]).wait()
                    pltpu.async_copy(rows[k], acc.at[dst_v.at[j0 + k]],
                                     ssem[k], add=True)
                for k in range(4):
                    pltpu.make_async_copy(
                        rows[k], acc.at[dst_v.at[0]], ssem[k]).wait()
                    nj = j0 + k + 4

                    @pl.when(nj < NBS)
                    def _():
                        pltpu.async_copy(tbl.at[src_v.at[nj]], rows[k],
                                         gsem[k])
                return carry

            lax.fori_loop(0, (NBS - 1) // 4, group, 0)
            pltpu.make_async_copy(tbl.at[src_v.at[0]], rows[0], gsem[0]).wait()
            pltpu.sync_copy(rows[0], acc.at[dst_v.at[NBS - 1]], add=True)

        plsc.subcore_barrier()

        @pl.when(s < WT)
        def _():
            pltpu.sync_copy(acc.at[pl.ds(s * RPW, RPW)],
                            out_hbm.at[pl.ds(q * N + s * RPW, RPW)])
        plsc.subcore_barrier()


RB = 64


@functools.partial(
    pl.kernel,
    out_type=jax.ShapeDtypeStruct((NW, 8, CW), _f32),
    mesh=_MESH,
    scratch_types=[
        pltpu.VMEM((16,), jnp.int32),
        pltpu.VMEM((16,), jnp.int32),
        pltpu.VMEM((RB, CW), _f32),
        pltpu.VMEM((8, CW), _f32),
    ],
)
def _segmax_kernel(h2_hbm, st_hbm, en_hbm, out_hbm, st_v, en_v, blk_v, res_v):
    c = lax.axis_index("c")
    s = lax.axis_index("s")
    wid = s * NC + c
    pltpu.sync_copy(st_hbm.at[wid], st_v)
    pltpu.sync_copy(en_hbm.at[wid], en_v)
    st_vec = st_v[pl.ds(0, 16)]
    en_vec = en_v[pl.ds(0, 16)]

    for k in range(8):
        tid = k * NW + wid
        q = tid // NUM_GRAPHS
        start = st_vec[k]
        end = en_vec[k]
        rb0 = 8 * (start // 8)
        nblk = lax.max((end - rb0 + RB - 1) // RB, 0)

        def block(t, accs):
            rb = jnp.minimum(rb0 + t * RB, N - RB)
            pltpu.sync_copy(h2_hbm.at[pl.ds(q * N + rb, RB)], blk_v)

            def row(r, accs):
                keep = (rb + r >= start) & (rb + r < end)
                return tuple(
                    jnp.where(keep,
                              jnp.maximum(accs[i], blk_v[r, pl.ds(i * 16, 16)]),
                              accs[i])
                    for i in range(CW // 16))

            return lax.fori_loop(0, RB, row, accs)

        neg = jnp.full((16,), -jnp.inf, _f32)
        accs = lax.fori_loop(0, nblk, block, (neg,) * (CW // 16))
        for i in range(CW // 16):
            res_v[k, pl.ds(i * 16, 16)] = accs[i]

    pltpu.sync_copy(res_v, out_hbm.at[wid])


BR = 1000
_GRID = N // BR

_cm_spec = pl.BlockSpec((NCH, BR, CW), lambda i: (0, i, 0))
_dinv_spec = pl.BlockSpec((BR, 1), lambda i: (i, 0))
_b_spec = pl.BlockSpec((NCH, 1, CW), lambda i: (0, 0, 0))
_w_spec = pl.BlockSpec((512, 512), lambda i: (0, 0))


def _deg_reduce_kernel(parts_ref, dinv_ref):
    deg = parts_ref[:, 0:1] + 1.0
    dinv_ref[...] = lax.rsqrt(deg)


def _deg_reduce(parts):
    return pl.pallas_call(
        _deg_reduce_kernel,
        in_specs=[pl.BlockSpec((N, DW), lambda: (0, 0))],
        out_specs=pl.BlockSpec((N, 1), lambda: (0, 0)),
        out_shape=jax.ShapeDtypeStruct((N, 1), _f32),
    )(parts)


def _input_kernel(x_ref, wesm_ref, besm_ref, nat_ref, embp_ref, waa_ref,
                  baa_ref, h_ref, xr_ref):
    xesm = jnp.dot(x_ref[...], wesm_ref[...],
                   preferred_element_type=_f32) + besm_ref[...]
    embw = jnp.dot(embp_ref[...], waa_ref[...], preferred_element_type=_f32)
    oh = (nat_ref[...] == lax.broadcasted_iota(jnp.int32, (BR, 32), 1)
          ).astype(_f32)
    xaa = jnp.dot(oh, embw, preferred_element_type=_f32) + baa_ref[...]
    h = jax.nn.relu(xaa + xesm)
    xr = jax.nn.relu(xesm)
    for q in range(NCH):
        h_ref[q] = h[:, q * CW:(q + 1) * CW]
        xr_ref[q] = xr[:, q * CW:(q + 1) * CW]


def _input_call(x, W_esm, b_esm, nat2, emb_p, W_aa, b_aa):
    cm = jax.ShapeDtypeStruct((NCH, N, CW), _f32)
    return pl.pallas_call(
        _input_kernel,
        grid=(_GRID,),
        in_specs=[
            pl.BlockSpec((BR, 1280), lambda i: (i, 0)),
            pl.BlockSpec((1280, 512), lambda i: (0, 0)),
            pl.BlockSpec((1, 512), lambda i: (0, 0)),
            pl.BlockSpec((BR, 1), lambda i: (i, 0)),
            pl.BlockSpec((32, 96), lambda i: (0, 0)),
            pl.BlockSpec((96, 512), lambda i: (0, 0)),
            pl.BlockSpec((1, 512), lambda i: (0, 0)),
        ],
        out_specs=[_cm_spec, _cm_spec],
        out_shape=[cm, cm],
    )(x, W_esm, b_esm.reshape(1, 512), nat2, emb_p, W_aa, b_aa.reshape(1, 512))


def _first_kernel(feat_ref, dinv_ref, w_ref, hp_ref):
    xb = jnp.concatenate([feat_ref[q] for q in range(NCH)], axis=-1)
    mm = jnp.dot(xb, w_ref[...], preferred_element_type=_f32) * dinv_ref[...]
    for q in range(NCH):
        hp_ref[q] = mm[:, q * CW:(q + 1) * CW]


def _first_mm(feat, dinv, W):
    return pl.pallas_call(
        _first_kernel,
        grid=(_GRID,),
        in_specs=[_cm_spec, _dinv_spec, _w_spec],
        out_specs=_cm_spec,
        out_shape=jax.ShapeDtypeStruct((NCH, N, CW), _f32),
    )(feat, dinv, W)


def _mid_kernel(agg_ref, hp_ref, res_ref, dinv_ref, b_ref, w_ref,
                h_ref, hpn_ref, *, has_res):
    dinv = dinv_ref[...]
    parts = []
    for q in range(NCH):
        t = jax.nn.relu(dinv * (agg_ref[q] + hp_ref[q]) + b_ref[q])
        if has_res:
            t = res_ref[q] + t
        h_ref[q] = t
        parts.append(t)
    xb = jnp.concatenate(parts, axis=-1)
    mm = jnp.dot(xb, w_ref[...], preferred_element_type=_f32) * dinv
    for q in range(NCH):
        hpn_ref[q] = mm[:, q * CW:(q + 1) * CW]


def _mid_mm(agg, hp, res, dinv, b, W):
    cm = jax.ShapeDtypeStruct((NCH, N, CW), _f32)
    has_res = res is not None
    in_specs = [_cm_spec, _cm_spec]
    args = [agg, hp]
    if has_res:
        in_specs.append(_cm_spec)
        args.append(res)
    in_specs += [_dinv_spec, _b_spec, _w_spec]
    args += [dinv, b.reshape(NCH, 1, CW), W]
    body = functools.partial(_mid_kernel, has_res=has_res)
    if not has_res:
        def body(agg_ref, hp_ref, dinv_ref, b_ref, w_ref, h_ref, hpn_ref):
            return _mid_kernel(agg_ref, hp_ref, None, dinv_ref, b_ref, w_ref,
                               h_ref, hpn_ref, has_res=False)
    return pl.pallas_call(
        body,
        grid=(_GRID,),
        in_specs=in_specs,
        out_specs=[_cm_spec, _cm_spec],
        out_shape=[cm, cm],
    )(*args)


def _few_kernel(agg_ref, hp_ref, res_ref, dinv_ref, b_ref, out_ref):
    dinv = dinv_ref[...]
    for q in range(NCH):
        t = jax.nn.relu(dinv * (agg_ref[q] + hp_ref[q]) + b_ref[q])
        out_ref[q] = res_ref[q] + t


def _final_ew(agg, hp, res, dinv, b):
    return pl.pallas_call(
        _few_kernel,
        grid=(_GRID,),
        in_specs=[_cm_spec, _cm_spec, _cm_spec, _dinv_spec, _b_spec],
        out_specs=_cm_spec,
        out_shape=jax.ShapeDtypeStruct((NCH, N, CW), _f32),
    )(agg, hp, res, dinv, b.reshape(NCH, 1, CW))


def _parts_to_g(p_ref):
    half0 = jnp.concatenate([p_ref[:, 2 * q, :] for q in range(NCH)], axis=1)
    half1 = jnp.concatenate([p_ref[:, 2 * q + 1, :] for q in range(NCH)],
                            axis=1)
    return jnp.concatenate([half0, half1], axis=0)


def _head_kernel(g1_ref, g3_ref, w1_ref, b1_ref, w2_ref, b2_ref, y_ref):
    g = 0.5 * _parts_to_g(g1_ref) + 0.5 * _parts_to_g(g3_ref)
    z = jax.nn.relu(jnp.dot(g, w1_ref[...], preferred_element_type=_f32)
                    + b1_ref[...])
    y = jnp.dot(z, w2_ref[...], preferred_element_type=_f32) + b2_ref[...]
    y_ref[...] = jax.nn.sigmoid(y)


def _head(g1p, g3p, W_r1, b_r1, W_r2, b_r2):
    full = lambda shape: pl.BlockSpec(shape, lambda: tuple(0 for _ in shape))
    return pl.pallas_call(
        _head_kernel,
        in_specs=[full((NW, 8, CW)), full((NW, 8, CW)),
                  full((512, 1024)), full((1, 1024)),
                  full((1024, OUT_DIM)), full((1, OUT_DIM))],
        out_specs=full((NUM_GRAPHS, OUT_DIM)),
        out_shape=jax.ShapeDtypeStruct((NUM_GRAPHS, OUT_DIM), _f32),
    )(g1p, g3p, W_r1, b_r1.reshape(1, 1024), W_r2, b_r2.reshape(1, OUT_DIM))


def kernel(native_x, x, edge_index, batch, emb, W_aa, b_aa, W_esm, b_esm,
           W_g0, b_g0, W_g1, b_g1, W_g2, b_g2, W_r1, b_r1, W_r2, b_r2):
    src = edge_index[0].astype(jnp.int32)
    dst = edge_index[1].astype(jnp.int32)
    src3 = src.reshape(NS, ST, NBS, B)
    dst3 = dst.reshape(NS, ST, NBS, B)

    deg_parts = _deg_kernel(dst3)
    dinv = _deg_reduce(deg_parts)

    emb_p = jnp.zeros((32, 96), _f32).at[:21].set(emb)
    h_cm, xr_cm = _input_call(x, W_esm, b_esm, native_x.reshape(N, 1).astype(jnp.int32),
                              emb_p, W_aa, b_aa)

    def spmm(hp_cm):
        out = _spmm_kernel(hp_cm.reshape(NCH * N, CW), src3, dst3)
        return out.reshape(NCH, N, CW)

    starts = jnp.searchsorted(
        batch.astype(jnp.int32),
        jnp.arange(NUM_GRAPHS + 1, dtype=jnp.int32)).astype(jnp.int32)
    tids = (jnp.arange(8)[None, :] * NW + jnp.arange(NW)[:, None])
    gof = tids % NUM_GRAPHS
    st_t = jnp.zeros((NW, 16), jnp.int32).at[:, :8].set(starts[gof])
    en_t = jnp.zeros((NW, 16), jnp.int32).at[:, :8].set(starts[gof + 1])

    def graphcnn(feat_cm):
        hp0 = _first_mm(feat_cm, dinv, W_g0)
        agg0 = spmm(hp0)
        h0, hp1 = _mid_mm(agg0, hp0, None, dinv, b_g0, W_g1)
        agg1 = spmm(hp1)
        h1, hp2 = _mid_mm(agg1, hp1, h0, dinv, b_g1, W_g2)
        agg2 = spmm(hp2)
        h2 = _final_ew(agg2, hp2, h1, dinv, b_g2)
        return _segmax_kernel(h2.reshape(NCH * N, CW), st_t, en_t)

    g1p = graphcnn(h_cm)
    g3p = graphcnn(xr_cm)
    return _head(g1p, g3p, W_r1, b_r1, W_r2, b_r2)
